# speedup vs baseline: 1.4457x; 1.4457x over previous
import jax
import jax.numpy as jnp
from jax.experimental import pallas as pl
from jax.experimental.pallas import tpu as pltpu

FEAT = 16
HIDDEN = 128
OUT_DIM = 64
BLOCK_R = 2048


def _mlp_block(ev_ref, w1_ref, b1_ref, w2_ref, b2_ref, out_ref):
    ev = ev_ref[...]
    x = ev[:, :FEAT]
    m = ev[:, FEAT:FEAT + 1]
    h = jnp.maximum(
        jnp.dot(x, w1_ref[...], preferred_element_type=jnp.float32)
        + b1_ref[...], 0.0)
    o = jnp.dot(h, w2_ref[...], preferred_element_type=jnp.float32) + b2_ref[...]
    full = jnp.concatenate([o, jnp.ones_like(m)], axis=1)
    out_ref[...] = jnp.where(m == 1.0, full, 0.0)


@jax.jit
def kernel(events, W1, b1, W2, b2):
    B, P, F = events.shape
    rows = B * P
    flat = events.reshape(rows, F)
    grid = rows // BLOCK_R
    out = pl.pallas_call(
        _mlp_block,
        grid=(grid,),
        in_specs=[
            pl.BlockSpec((BLOCK_R, F), lambda i: (i, 0)),
            pl.BlockSpec((FEAT, HIDDEN), lambda i: (0, 0)),
            pl.BlockSpec((1, HIDDEN), lambda i: (0, 0)),
            pl.BlockSpec((HIDDEN, OUT_DIM), lambda i: (0, 0)),
            pl.BlockSpec((1, OUT_DIM), lambda i: (0, 0)),
        ],
        out_specs=pl.BlockSpec((BLOCK_R, OUT_DIM + 1), lambda i: (i, 0)),
        out_shape=jax.ShapeDtypeStruct((rows, OUT_DIM + 1), jnp.float32),
        compiler_params=pltpu.CompilerParams(
            dimension_semantics=("arbitrary",),
        ),
    )(flat, W1, b1.reshape(1, HIDDEN), W2, b2.reshape(1, OUT_DIM))
    return out.reshape(B, P, OUT_DIM + 1)


# speedup vs baseline: 1.5141x; 1.0473x over previous
import jax
import jax.numpy as jnp
from jax.experimental import pallas as pl
from jax.experimental.pallas import tpu as pltpu

FEAT = 16
HIDDEN = 128
OUT_DIM = 64
BLOCK_R = 2048


BLOCK_E = 16


def _mlp_block(ev_ref, w1_ref, b1_ref, w2_ref, b2_ref, out_ref):
    be, p, f = ev_ref.shape
    ev = ev_ref[...].reshape(be * p, f)
    x = ev[:, :FEAT]
    m = ev[:, FEAT:FEAT + 1]
    h = jnp.maximum(
        jnp.dot(x, w1_ref[...], preferred_element_type=jnp.float32)
        + b1_ref[...], 0.0)
    o = jnp.dot(h, w2_ref[...], preferred_element_type=jnp.float32) + b2_ref[...]
    full = jnp.concatenate([o, jnp.ones_like(m)], axis=1)
    out_ref[...] = jnp.where(m == 1.0, full, 0.0).reshape(be, p, OUT_DIM + 1)


@jax.jit
def kernel(events, W1, b1, W2, b2):
    B, P, F = events.shape
    grid = B // BLOCK_E
    return pl.pallas_call(
        _mlp_block,
        grid=(grid,),
        in_specs=[
            pl.BlockSpec((BLOCK_E, P, F), lambda i: (i, 0, 0)),
            pl.BlockSpec((FEAT, HIDDEN), lambda i: (0, 0)),
            pl.BlockSpec((1, HIDDEN), lambda i: (0, 0)),
            pl.BlockSpec((HIDDEN, OUT_DIM), lambda i: (0, 0)),
            pl.BlockSpec((1, OUT_DIM), lambda i: (0, 0)),
        ],
        out_specs=pl.BlockSpec((BLOCK_E, P, OUT_DIM + 1), lambda i: (i, 0, 0)),
        out_shape=jax.ShapeDtypeStruct((B, P, OUT_DIM + 1), jnp.float32),
        compiler_params=pltpu.CompilerParams(
            dimension_semantics=("arbitrary",),
        ),
    )(events, W1, b1.reshape(1, HIDDEN), W2, b2.reshape(1, OUT_DIM))


# speedup vs baseline: 6.2418x; 4.1225x over previous
import jax
import jax.numpy as jnp
from jax.experimental import pallas as pl
from jax.experimental.pallas import tpu as pltpu

FEAT = 16
HIDDEN = 128
OUT_DIM = 64
LANES = 512
PSUB = 8
N = PSUB * LANES


def _mlp_block(ev_ref, w1_ref, b1_ref, w2_ref, b2_ref, out_ref):
    ev = ev_ref[...]
    x = ev[:FEAT].reshape(FEAT, N)
    m = ev[FEAT:].reshape(1, N)
    h = jax.lax.dot_general(
        w1_ref[...], x, (((0,), (0,)), ((), ())),
        preferred_element_type=jnp.float32)
    h = jnp.maximum(h + b1_ref[...], 0.0)
    o = jax.lax.dot_general(
        w2_ref[...], h, (((0,), (0,)), ((), ())),
        preferred_element_type=jnp.float32)
    o = o + b2_ref[...]
    full = jnp.concatenate([o, jnp.ones_like(m)], axis=0)
    res = jnp.where(m == 1.0, full, 0.0)
    out_ref[...] = res.reshape(OUT_DIM + 1, PSUB, LANES)


@jax.jit
def kernel(events, W1, b1, W2, b2):
    B, P, F = events.shape
    ev_t = jnp.transpose(events, (2, 1, 0))
    out_t = pl.pallas_call(
        _mlp_block,
        grid=(P // PSUB, B // LANES),
        in_specs=[
            pl.BlockSpec((F, PSUB, LANES), lambda j, i: (0, j, i)),
            pl.BlockSpec((FEAT, HIDDEN), lambda j, i: (0, 0)),
            pl.BlockSpec((HIDDEN, 1), lambda j, i: (0, 0)),
            pl.BlockSpec((HIDDEN, OUT_DIM), lambda j, i: (0, 0)),
            pl.BlockSpec((OUT_DIM, 1), lambda j, i: (0, 0)),
        ],
        out_specs=pl.BlockSpec((OUT_DIM + 1, PSUB, LANES), lambda j, i: (0, j, i)),
        out_shape=jax.ShapeDtypeStruct((OUT_DIM + 1, P, B), jnp.float32),
        compiler_params=pltpu.CompilerParams(
            dimension_semantics=("arbitrary", "arbitrary"),
        ),
    )(ev_t, W1, b1.reshape(HIDDEN, 1), W2, b2.reshape(OUT_DIM, 1))
    return jnp.transpose(out_t, (2, 1, 0))


# speedup vs baseline: 7.5631x; 1.2117x over previous
import jax
import jax.numpy as jnp
from jax.experimental import pallas as pl
from jax.experimental.pallas import tpu as pltpu

FEAT = 16
HIDDEN = 128
OUT_DIM = 64
LANES = 1024
PSUB = 8
N = PSUB * LANES


def _mlp_block(ev_ref, w1_ref, b1_ref, w2_ref, b2_ref, out_ref):
    ev = ev_ref[...]
    x = ev[:FEAT].reshape(FEAT, N)
    m = ev[FEAT:].reshape(1, N)
    h = jax.lax.dot_general(
        w1_ref[...], x, (((0,), (0,)), ((), ())),
        preferred_element_type=jnp.float32)
    h = jnp.maximum(h + b1_ref[...], 0.0)
    o = jax.lax.dot_general(
        w2_ref[...], h, (((0,), (0,)), ((), ())),
        preferred_element_type=jnp.float32)
    o = o + b2_ref[...]
    full = jnp.concatenate([o, jnp.ones_like(m)], axis=0)
    res = jnp.where(m == 1.0, full, 0.0)
    out_ref[...] = res.reshape(OUT_DIM + 1, PSUB, LANES)


@jax.jit
def kernel(events, W1, b1, W2, b2):
    B, P, F = events.shape
    ev_t = jnp.transpose(events, (2, 1, 0))
    out_t = pl.pallas_call(
        _mlp_block,
        grid=(P // PSUB, B // LANES),
        in_specs=[
            pl.BlockSpec((F, PSUB, LANES), lambda j, i: (0, j, i)),
            pl.BlockSpec((FEAT, HIDDEN), lambda j, i: (0, 0)),
            pl.BlockSpec((HIDDEN, 1), lambda j, i: (0, 0)),
            pl.BlockSpec((HIDDEN, OUT_DIM), lambda j, i: (0, 0)),
            pl.BlockSpec((OUT_DIM, 1), lambda j, i: (0, 0)),
        ],
        out_specs=pl.BlockSpec((OUT_DIM + 1, PSUB, LANES), lambda j, i: (0, j, i)),
        out_shape=jax.ShapeDtypeStruct((OUT_DIM + 1, P, B), jnp.float32),
        compiler_params=pltpu.CompilerParams(
            dimension_semantics=("arbitrary", "arbitrary"),
        ),
    )(ev_t, W1, b1.reshape(HIDDEN, 1), W2, b2.reshape(OUT_DIM, 1))
    return jnp.transpose(out_t, (2, 1, 0))


# speedup vs baseline: 7.9487x; 1.0510x over previous
import jax
import jax.numpy as jnp
from jax.experimental import pallas as pl
from jax.experimental.pallas import tpu as pltpu

FEAT = 16
HIDDEN = 128
OUT_DIM = 64
LANES = 2048
PSUB = 8
N = PSUB * LANES


def _mlp_block(ev_ref, w1_ref, b1_ref, w2_ref, b2_ref, out_ref):
    ev = ev_ref[...]
    x = ev[:FEAT].reshape(FEAT, N)
    m = ev[FEAT:].reshape(1, N)
    h = jax.lax.dot_general(
        w1_ref[...], x, (((0,), (0,)), ((), ())),
        preferred_element_type=jnp.float32)
    h = jnp.maximum(h + b1_ref[...], 0.0)
    o = jax.lax.dot_general(
        w2_ref[...], h, (((0,), (0,)), ((), ())),
        preferred_element_type=jnp.float32)
    o = o + b2_ref[...]
    full = jnp.concatenate([o, jnp.ones_like(m)], axis=0)
    res = jnp.where(m == 1.0, full, 0.0)
    out_ref[...] = res.reshape(OUT_DIM + 1, PSUB, LANES)


@jax.jit
def kernel(events, W1, b1, W2, b2):
    B, P, F = events.shape
    ev_t = jnp.transpose(events, (2, 1, 0))
    out_t = pl.pallas_call(
        _mlp_block,
        grid=(P // PSUB, B // LANES),
        in_specs=[
            pl.BlockSpec((F, PSUB, LANES), lambda j, i: (0, j, i)),
            pl.BlockSpec((FEAT, HIDDEN), lambda j, i: (0, 0)),
            pl.BlockSpec((HIDDEN, 1), lambda j, i: (0, 0)),
            pl.BlockSpec((HIDDEN, OUT_DIM), lambda j, i: (0, 0)),
            pl.BlockSpec((OUT_DIM, 1), lambda j, i: (0, 0)),
        ],
        out_specs=pl.BlockSpec((OUT_DIM + 1, PSUB, LANES), lambda j, i: (0, j, i)),
        out_shape=jax.ShapeDtypeStruct((OUT_DIM + 1, P, B), jnp.float32),
        compiler_params=pltpu.CompilerParams(
            dimension_semantics=("arbitrary", "arbitrary"),
        ),
    )(ev_t, W1, b1.reshape(HIDDEN, 1), W2, b2.reshape(OUT_DIM, 1))
    return jnp.transpose(out_t, (2, 1, 0))


# speedup vs baseline: 8.1301x; 1.0228x over previous
import jax
import jax.numpy as jnp
from jax.experimental import pallas as pl
from jax.experimental.pallas import tpu as pltpu

FEAT = 16
HIDDEN = 128
OUT_DIM = 64
LANES = 4096
PSUB = 8
N = PSUB * LANES


def _mlp_block(ev_ref, w1_ref, b1_ref, w2_ref, b2_ref, out_ref):
    ev = ev_ref[...]
    x = ev[:FEAT].reshape(FEAT, N)
    m = ev[FEAT:].reshape(1, N)
    h = jax.lax.dot_general(
        w1_ref[...], x, (((0,), (0,)), ((), ())),
        preferred_element_type=jnp.float32)
    h = jnp.maximum(h + b1_ref[...], 0.0)
    o = jax.lax.dot_general(
        w2_ref[...], h, (((0,), (0,)), ((), ())),
        preferred_element_type=jnp.float32)
    o = o + b2_ref[...]
    full = jnp.concatenate([o, jnp.ones_like(m)], axis=0)
    res = jnp.where(m == 1.0, full, 0.0)
    out_ref[...] = res.reshape(OUT_DIM + 1, PSUB, LANES)


@jax.jit
def kernel(events, W1, b1, W2, b2):
    B, P, F = events.shape
    ev_t = jnp.transpose(events, (2, 1, 0))
    out_t = pl.pallas_call(
        _mlp_block,
        grid=(P // PSUB, B // LANES),
        in_specs=[
            pl.BlockSpec((F, PSUB, LANES), lambda j, i: (0, j, i)),
            pl.BlockSpec((FEAT, HIDDEN), lambda j, i: (0, 0)),
            pl.BlockSpec((HIDDEN, 1), lambda j, i: (0, 0)),
            pl.BlockSpec((HIDDEN, OUT_DIM), lambda j, i: (0, 0)),
            pl.BlockSpec((OUT_DIM, 1), lambda j, i: (0, 0)),
        ],
        out_specs=pl.BlockSpec((OUT_DIM + 1, PSUB, LANES), lambda j, i: (0, j, i)),
        out_shape=jax.ShapeDtypeStruct((OUT_DIM + 1, P, B), jnp.float32),
        compiler_params=pltpu.CompilerParams(
            dimension_semantics=("arbitrary", "arbitrary"),
        ),
    )(ev_t, W1, b1.reshape(HIDDEN, 1), W2, b2.reshape(OUT_DIM, 1))
    return jnp.transpose(out_t, (2, 1, 0))


# speedup vs baseline: 8.1328x; 1.0003x over previous
import jax
import jax.numpy as jnp
from jax.experimental import pallas as pl
from jax.experimental.pallas import tpu as pltpu

FEAT = 16
HIDDEN = 128
OUT_DIM = 64
LANES = 4096
PSUB = 8
N = PSUB * LANES


def _mlp_block(ev_ref, w1_ref, b1_ref, w2_ref, b2_ref, out_ref):
    ev = ev_ref[...]
    x = ev[:FEAT].reshape(FEAT, N)
    m = ev[FEAT:].reshape(1, N)
    h = jax.lax.dot_general(
        w1_ref[...], x, (((0,), (0,)), ((), ())),
        preferred_element_type=jnp.float32)
    h = jnp.maximum(h + b1_ref[...], 0.0)
    o = jax.lax.dot_general(
        w2_ref[...], h, (((0,), (0,)), ((), ())),
        preferred_element_type=jnp.float32)
    o = o + b2_ref[...]
    full = jnp.concatenate([o, jnp.ones_like(m)], axis=0)
    res = jnp.where(m == 1.0, full, 0.0)
    out_ref[...] = res.reshape(OUT_DIM + 1, PSUB, LANES)


@jax.jit
def kernel(events, W1, b1, W2, b2):
    B, P, F = events.shape
    ev_t = jnp.transpose(events, (2, 1, 0))
    out_t = pl.pallas_call(
        _mlp_block,
        grid=(P // PSUB, B // LANES),
        in_specs=[
            pl.BlockSpec((F, PSUB, LANES), lambda j, i: (0, j, i)),
            pl.BlockSpec((FEAT, HIDDEN), lambda j, i: (0, 0)),
            pl.BlockSpec((HIDDEN, 1), lambda j, i: (0, 0)),
            pl.BlockSpec((HIDDEN, OUT_DIM), lambda j, i: (0, 0)),
            pl.BlockSpec((OUT_DIM, 1), lambda j, i: (0, 0)),
        ],
        out_specs=pl.BlockSpec((OUT_DIM + 1, PSUB, LANES), lambda j, i: (0, j, i)),
        out_shape=jax.ShapeDtypeStruct((OUT_DIM + 1, P, B), jnp.float32),
        compiler_params=pltpu.CompilerParams(
            dimension_semantics=("parallel", "parallel"),
        ),
    )(ev_t, W1, b1.reshape(HIDDEN, 1), W2, b2.reshape(OUT_DIM, 1))
    return jnp.transpose(out_t, (2, 1, 0))
